# baseline (speedup 1.0000x reference)
import functools

import jax
import jax.numpy as jnp
from jax import lax
from jax.experimental import pallas as pl
from jax.experimental.pallas import tpu as pltpu
from jax.experimental.pallas import tpu_sc as plsc

NEG_SLOPE = 0.2
NC = 2
NS = 16
NW = NC * NS
LANES = 16


def _tc_pre(x_src, x_dst, Wl_b, Wr_b, Wl_f, Wr_f):
    n, d = x_src.shape
    out = Wl_b.shape[1]

    def body(xs, xd, wlb, wrb, wlf, wrf, hlb, hrb, hlf, hrf):
        hlb[...] = jnp.dot(xs[...], wlb[...], preferred_element_type=jnp.float32)
        hrb[...] = jnp.dot(xd[...], wrb[...], preferred_element_type=jnp.float32)
        hlf[...] = jnp.dot(xd[...], wlf[...], preferred_element_type=jnp.float32)
        hrf[...] = jnp.dot(xs[...], wrf[...], preferred_element_type=jnp.float32)

    o = jax.ShapeDtypeStruct((n, out), jnp.float32)
    return pl.pallas_call(body, out_shape=[o, o, o, o])(
        x_src, x_dst, Wl_b, Wr_b, Wl_f, Wr_f)


def _tc_edge(ea, We_b, We_f):
    e, de = ea.shape
    out = We_b.shape[1]
    be = 8000
    grid = e // be

    def body(ea_ref, wb_ref, wf_ref, ob_ref, of_ref):
        ob_ref[...] = jnp.dot(ea_ref[...], wb_ref[...], preferred_element_type=jnp.float32)
        of_ref[...] = jnp.dot(ea_ref[...], wf_ref[...], preferred_element_type=jnp.float32)

    o = jax.ShapeDtypeStruct((e, out), jnp.float32)
    return pl.pallas_call(
        body,
        grid=(grid,),
        in_specs=[
            pl.BlockSpec((be, de), lambda i: (i, 0)),
            pl.BlockSpec((de, out), lambda i: (0, 0)),
            pl.BlockSpec((de, out), lambda i: (0, 0)),
        ],
        out_specs=[pl.BlockSpec((be, out), lambda i: (i, 0))] * 2,
        out_shape=[o, o],
    )(ea, We_b, We_f)


def _tc_fin(num, den, b):
    n, out = den.shape[1], num.shape[2]

    def body(num_ref, den_ref, b_ref, o_ref):
        s = num_ref[0, :n, :] + num_ref[1, :n, :]
        d = jnp.sum(den_ref[...], axis=0) + 1e-16
        o_ref[...] = s / d[:, None] + b_ref[...]

    return pl.pallas_call(
        body, out_shape=jax.ShapeDtypeStruct((n, out), jnp.float32))(num, den, b)


def _sc_pass(hl, hr, he, src, dst, att):
    n, out = hl.shape
    e = src.shape[0]
    epw = e // NW
    c = 32
    nchunk = epw // c
    rem = epw - nchunk * c
    nj = out // LANES
    n_pad = ((n + NS * 128 - 1) // (NS * 128)) * (NS * 128)
    rows_per_tile = n_pad // NS
    wb_full = rows_per_tile // c
    wb_rem = rows_per_tile - wb_full * c
    assert epw * NW == e and rem % LANES == 0 and rem < c
    assert nchunk % 2 == 0 and nchunk >= 4 and wb_rem % 8 == 0

    mesh = plsc.VectorSubcoreMesh(core_axis_name="c", subcore_axis_name="s")

    @functools.partial(
        pl.kernel,
        out_type=[jax.ShapeDtypeStruct((NC, n_pad, out), jnp.float32),
                  jax.ShapeDtypeStruct((NW, n), jnp.float32)],
        mesh=mesh,
        compiler_params=pltpu.CompilerParams(needs_layout_passes=False),
        scratch_types=[
            pltpu.VMEM((c,), jnp.int32),
            pltpu.VMEM((c,), jnp.int32),
            pltpu.VMEM((c,), jnp.int32),
            pltpu.VMEM((c,), jnp.int32),
            pltpu.VMEM((c,), jnp.int32),
            pltpu.VMEM((LANES,), jnp.int32),
            pltpu.VMEM((c, out), jnp.float32),
            pltpu.VMEM((c, out), jnp.float32),
            pltpu.VMEM((c, out), jnp.float32),
            pltpu.VMEM((c, out), jnp.float32),
            pltpu.VMEM((c, out), jnp.float32),
            pltpu.VMEM((LANES * LANES,), jnp.float32),
            pltpu.VMEM((out,), jnp.float32),
            pltpu.VMEM((n,), jnp.float32),
            pltpu.VMEM_SHARED((n_pad, out), jnp.float32),
            pltpu.SemaphoreType.DMA,
            pltpu.SemaphoreType.DMA,
            pltpu.SemaphoreType.DMA,
            pltpu.SemaphoreType.DMA,
            pltpu.SemaphoreType.DMA,
            pltpu.SemaphoreType.DMA,
            pltpu.SemaphoreType.DMA,
            pltpu.SemaphoreType.DMA,
            pltpu.SemaphoreType.DMA,
        ],
    )
    def k(hl_h, hr_h, he_h, src_h, dst_h, att_h, num_h, den_h,
          ixs0, ixd0, ixs1, ixd1, sidx, sidxr,
          gs0, gs1, gx0, gx1, wgs, accb, attv, dloc, nums,
          sgs0, sgs1, sgx0, sgx1, she0, she1, six0, six1, ssc):
        cid = lax.axis_index("c")
        sid = lax.axis_index("s")
        wid = cid * NS + sid
        ebase = wid * epw
        z16 = jnp.zeros((LANES,), jnp.float32)
        IXS = (ixs0, ixs1)
        IXD = (ixd0, ixd1)
        GS = (gs0, gs1)
        GX = (gx0, gx1)
        SGS = (sgs0, sgs1)
        SGX = (sgx0, sgx1)
        SHE = (she0, she1)
        SIX = (six0, six1)

        def zero_gs0(i, carry):
            for j in range(nj):
                gs0[i, pl.ds(j * LANES, LANES)] = z16
            return carry
        lax.fori_loop(0, c, zero_gs0, 0)

        def zero_dloc(i, carry):
            dloc[pl.ds(pl.multiple_of(i * LANES, LANES), LANES)] = z16
            return carry
        lax.fori_loop(0, n // LANES, zero_dloc, 0)

        for kb in range(wb_full):
            rb = sid * rows_per_tile + kb * c
            pltpu.sync_copy(gs0, nums.at[pl.ds(rb, c)])
        if wb_rem:
            rb = sid * rows_per_tile + wb_full * c
            pltpu.sync_copy(gs0.at[pl.ds(0, wb_rem)], nums.at[pl.ds(rb, wb_rem)])
        pltpu.sync_copy(att_h, attv)
        plsc.subcore_barrier()

        def issue_idx(j, S):
            b = pl.multiple_of(ebase + j * c, 8)
            pltpu.async_copy(src_h.at[pl.ds(b, c)], IXS[S], SIX[S])
            pltpu.async_copy(dst_h.at[pl.ds(b, c)], IXD[S], SIX[S])

        def wait_idx(S):
            pltpu.make_async_copy(src_h.at[pl.ds(0, c)], IXS[S], SIX[S]).wait()
            pltpu.make_async_copy(dst_h.at[pl.ds(0, c)], IXD[S], SIX[S]).wait()

        def alpha_scale_group(gsb, gxb, ixdb, goff):
            for ee in range(LANES):
                r = goff + ee
                acc = None
                for j in range(nj):
                    sl = pl.ds(j * LANES, LANES)
                    m = gsb[r, sl] + gxb[r, sl]
                    m = jnp.maximum(m, NEG_SLOPE * m)
                    t = m * attv[sl]
                    acc = t if acc is None else acc + t
                accb[pl.ds(ee * LANES, LANES)] = acc
            rowb = lax.iota(jnp.int32, LANES) * LANES
            al = None
            for j in range(LANES):
                t = plsc.load_gather(accb, [rowb + j])
                al = t if al is None else al + t
            ex = jnp.exp(al)
            dvec = ixdb[pl.ds(goff, LANES)]
            plsc.addupdate_scatter(dloc, [dvec], ex)
            for ee in range(LANES):
                r = goff + ee
                ei = ex[ee]
                for j in range(nj):
                    sl = pl.ds(j * LANES, LANES)
                    wgs[r, sl] = gsb[r, sl] * ei

        def process(j, S, has_prev, has_next, has_next2):
            O = 1 - S
            pltpu.make_async_copy(hl_h.at[IXS[S]], GS[S], SGS[S]).wait()
            pltpu.make_async_copy(hr_h.at[IXD[S]], GX[S], SGX[S]).wait()

            def start_next():
                wait_idx(O)
                b = pl.multiple_of(ebase + (j + 1) * c, 8)
                pltpu.async_copy(he_h.at[pl.ds(b, c)], GX[O], SHE[O])
                pltpu.async_copy(hl_h.at[IXS[O]], GS[O], SGS[O])
            if has_next is True:
                start_next()
            else:
                pl.when(has_next)(start_next)

            def drain_prev():
                pltpu.make_async_copy(wgs, nums.at[sidx], ssc).wait()
            if has_prev is True:
                drain_prev()
            else:
                pl.when(has_prev)(drain_prev)

            for q in range(c // LANES):
                sl = pl.ds(q * LANES, LANES)
                sidx[sl] = IXD[S][sl]
            for gi in range(c // LANES):
                alpha_scale_group(GS[S], GX[S], IXD[S], gi * LANES)

            def fetch_idx2():
                issue_idx(j + 2, S)
            if has_next2 is True:
                fetch_idx2()
            else:
                pl.when(has_next2)(fetch_idx2)

            def add_next_gather():
                pltpu.make_async_copy(he_h.at[pl.ds(0, c)], GX[O], SHE[O]).wait()
                pltpu.async_copy(hr_h.at[IXD[O]], GX[O], SGX[O], add=True)
            if has_next is True:
                add_next_gather()
            else:
                pl.when(has_next)(add_next_gather)

            pltpu.async_copy(wgs, nums.at[sidx], ssc, add=True)

        b0 = pl.multiple_of(ebase, 8)
        pltpu.sync_copy(src_h.at[pl.ds(b0, c)], ixs0)
        pltpu.sync_copy(dst_h.at[pl.ds(b0, c)], ixd0)
        pltpu.sync_copy(he_h.at[pl.ds(b0, c)], gx0)
        pltpu.async_copy(hl_h.at[ixs0], gs0, sgs0)
        pltpu.async_copy(hr_h.at[ixd0], gx0, sgx0, add=True)
        issue_idx(1, 1)

        last = nchunk // 2 - 1

        def pair_body(kk, carry):
            process(2 * kk, 0, kk > 0, True, kk < last)
            process(2 * kk + 1, 1, True, kk < last, kk < last)
            return carry
        lax.fori_loop(0, nchunk // 2, pair_body, 0)
        pltpu.make_async_copy(wgs, nums.at[sidx], ssc).wait()

        if rem:
            br = pl.multiple_of(ebase + nchunk * c, 8)
            pltpu.sync_copy(src_h.at[pl.ds(br, rem)], ixs0.at[pl.ds(0, rem)])
            pltpu.sync_copy(dst_h.at[pl.ds(br, rem)], ixd0.at[pl.ds(0, rem)])
            pltpu.sync_copy(he_h.at[pl.ds(br, rem)], gx0.at[pl.ds(0, rem)])
            pltpu.async_copy(hl_h.at[ixs0.at[pl.ds(0, rem)]],
                             gs0.at[pl.ds(0, rem)], sgs0).wait()
            pltpu.async_copy(hr_h.at[ixd0.at[pl.ds(0, rem)]],
                             gx0.at[pl.ds(0, rem)], sgx0, add=True).wait()
            for gi in range(rem // LANES):
                alpha_scale_group(gs0, gx0, ixd0, gi * LANES)
            for q in range(rem // LANES):
                sl = pl.ds(q * LANES, LANES)
                sidxr[sl] = ixd0[sl]
            pltpu.async_copy(wgs.at[pl.ds(0, rem)], nums.at[sidxr],
                             ssc, add=True).wait()

        plsc.subcore_barrier()
        for kb in range(wb_full):
            rb = sid * rows_per_tile + kb * c
            pltpu.sync_copy(nums.at[pl.ds(rb, c)],
                            num_h.at[cid, pl.ds(rb, c)])
        if wb_rem:
            rb = sid * rows_per_tile + wb_full * c
            pltpu.sync_copy(nums.at[pl.ds(rb, wb_rem)],
                            num_h.at[cid, pl.ds(rb, wb_rem)])
        pltpu.sync_copy(dloc, den_h.at[wid])

    return k(hl, hr, he, src, dst, att)


def kernel(x_src, x_dst, edge_index, edge_attr,
           Wl_f, Wr_f, We_f, att_f, b_f,
           Wl_b, Wr_b, We_b, att_b, b_b):
    src = edge_index[0]
    dst = edge_index[1]
    hlb, hrb, hlf, hrf = _tc_pre(x_src, x_dst, Wl_b, Wr_b, Wl_f, Wr_f)
    heb, hef = _tc_edge(edge_attr, We_b, We_f)
    numb, denb = _sc_pass(hlb, hrb, heb, src, dst, att_b)
    numf, denf = _sc_pass(hlf, hrf, hef, dst, src, att_f)
    out_bwd = _tc_fin(numb, denb, b_b)
    out_fwd = _tc_fin(numf, denf, b_f)
    return (out_bwd, out_fwd)
